# baseline (speedup 1.0000x reference)
import functools

import jax
import jax.numpy as jnp
from jax import lax
from jax.experimental import pallas as pl
from jax.experimental.pallas import tpu as pltpu
from jax.experimental.pallas import tpu_sc as plsc

BATCH = 16384
DIM = 64
HALF = DIM // 2
LANES = 16
NUM_WORKERS = 32
BPW = BATCH // NUM_WORKERS
GROUPS = BPW // LANES

_mesh = plsc.VectorSubcoreMesh(core_axis_name="c", subcore_axis_name="s")


@functools.partial(
    pl.kernel,
    mesh=_mesh,
    out_type=jax.ShapeDtypeStruct((BATCH,), jnp.float32),
    scratch_types=[
        pltpu.VMEM((BPW,), jnp.int32),
        pltpu.VMEM((BPW,), jnp.int32),
        pltpu.VMEM((BPW,), jnp.int32),
        pltpu.VMEM((BPW, DIM), jnp.float32),
        pltpu.VMEM((BPW, DIM), jnp.float32),
        pltpu.VMEM((BPW, DIM), jnp.float32),
        pltpu.VMEM((BPW,), jnp.float32),
        pltpu.SemaphoreType.DMA,
    ],
    compiler_params=pltpu.CompilerParams(use_tc_tiling_on_sc=False,
                                         needs_layout_passes=False),
)
def _kge_score(ent_hbm, rel_hbm, hidx_hbm, tidx_hbm, ridx_hbm, out_hbm,
               hidx_v, tidx_v, ridx_v, hrow, trow, rrow, outv, sem):
    wid = lax.axis_index("s") * 2 + lax.axis_index("c")
    base = wid * BPW

    pltpu.sync_copy(hidx_hbm.at[pl.ds(base, BPW)], hidx_v)
    pltpu.sync_copy(tidx_hbm.at[pl.ds(base, BPW)], tidx_v)
    pltpu.sync_copy(ridx_hbm.at[pl.ds(base, BPW)], ridx_v)

    cph = pltpu.async_copy(ent_hbm.at[hidx_v], hrow, sem)
    cpt = pltpu.async_copy(ent_hbm.at[tidx_v], trow, sem)
    cpr = pltpu.async_copy(rel_hbm.at[ridx_v], rrow, sem)
    cph.wait()
    cpt.wait()
    cpr.wait()

    lane_iota = lax.iota(jnp.int32, LANES)

    def group_body(g, carry):
        rows = g * LANES + lane_iota

        def dim_body(d, acc):
            cre = jnp.broadcast_to(d, (LANES,))
            cim = cre + HALF
            re_h = plsc.load_gather(hrow, [rows, cre])
            im_h = plsc.load_gather(hrow, [rows, cim])
            re_r = plsc.load_gather(rrow, [rows, cre])
            im_r = plsc.load_gather(rrow, [rows, cim])
            re_t = plsc.load_gather(trow, [rows, cre])
            im_t = plsc.load_gather(trow, [rows, cim])
            return acc + (re_h * (re_r * re_t + im_r * im_t)
                          + im_h * (re_r * im_t - im_r * re_t))

        acc = lax.fori_loop(0, HALF, dim_body,
                            jnp.zeros((LANES,), jnp.float32), unroll=4)
        outv[pl.ds(g * LANES, LANES)] = acc
        return carry

    lax.fori_loop(0, GROUPS, group_body, 0)

    pltpu.sync_copy(outv, out_hbm.at[pl.ds(base, BPW)])


def kernel(head_index, relation_index, tail_index, time_index,
           entity_embedding, relation_embedding, time_embedding,
           aux_embedding):
    del time_index, time_embedding, aux_embedding
    return _kge_score(entity_embedding, relation_embedding,
                      head_index.astype(jnp.int32),
                      tail_index.astype(jnp.int32),
                      relation_index.astype(jnp.int32))


# speedup vs baseline: 1.5521x; 1.5521x over previous
import functools

import jax
import jax.numpy as jnp
from jax import lax
from jax.experimental import pallas as pl
from jax.experimental.pallas import tpu as pltpu
from jax.experimental.pallas import tpu_sc as plsc

BATCH = 16384
DIM = 64
HALF = DIM // 2
LANES = 16
NREL = 230
NUM_WORKERS = 32
BPW = BATCH // NUM_WORKERS
NCH = 2
CH = BPW // NCH
K = 16
GROUPS = CH // LANES

_mesh = plsc.VectorSubcoreMesh(core_axis_name="c", subcore_axis_name="s")


@functools.partial(
    pl.kernel,
    mesh=_mesh,
    out_type=jax.ShapeDtypeStruct((BATCH,), jnp.float32),
    scratch_types=[
        pltpu.VMEM((BPW,), jnp.int32),
        pltpu.VMEM((BPW,), jnp.int32),
        pltpu.VMEM((BPW,), jnp.int32),
        pltpu.VMEM((CH, DIM), jnp.float32),
        pltpu.VMEM((CH, DIM), jnp.float32),
        pltpu.VMEM((NREL, DIM), jnp.float32),
        pltpu.VMEM((BPW,), jnp.float32),
        pltpu.SemaphoreType.DMA,
    ],
    compiler_params=pltpu.CompilerParams(needs_layout_passes=False),
)
def _kge_score(ent_hbm, rel_hbm, hidx_hbm, tidx_hbm, ridx_hbm, out_hbm,
               hsm, tsm, ridx_v, hrow, trow, rtab, outv, sem):
    wid = lax.axis_index("s") * 2 + lax.axis_index("c")
    base = wid * BPW

    pltpu.sync_copy(hidx_hbm.at[pl.ds(base, BPW)], hsm)
    pltpu.sync_copy(tidx_hbm.at[pl.ds(base, BPW)], tsm)
    pltpu.sync_copy(ridx_hbm.at[pl.ds(base, BPW)], ridx_v)
    pltpu.sync_copy(rel_hbm, rtab)

    lane_iota = lax.iota(jnp.int32, LANES)

    for c in range(NCH):
        cbase = c * CH

        def burst_body(j, carry):
            i0 = j * K
            hvec = hsm[pl.ds(cbase + i0, K)]
            tvec = tsm[pl.ds(cbase + i0, K)]
            copies = []
            for k in range(K):
                i = i0 + k
                hr = hvec[k]
                tr = tvec[k]
                copies.append(pltpu.async_copy(
                    ent_hbm.at[pl.ds(hr, 1), :], hrow.at[pl.ds(i, 1), :], sem))
                copies.append(pltpu.async_copy(
                    ent_hbm.at[pl.ds(tr, 1), :], trow.at[pl.ds(i, 1), :], sem))
            for cp in copies:
                cp.wait()
            return carry

        lax.fori_loop(0, CH // K, burst_body, 0)

        def group_body(g, carry):
            rows = g * LANES + lane_iota
            rrows = plsc.load_gather(ridx_v, [cbase + rows])

            def dim_body(d, acc):
                cre = jnp.broadcast_to(d, (LANES,))
                cim = cre + HALF
                re_h = plsc.load_gather(hrow, [rows, cre])
                im_h = plsc.load_gather(hrow, [rows, cim])
                re_r = plsc.load_gather(rtab, [rrows, cre])
                im_r = plsc.load_gather(rtab, [rrows, cim])
                re_t = plsc.load_gather(trow, [rows, cre])
                im_t = plsc.load_gather(trow, [rows, cim])
                return acc + (re_h * (re_r * re_t + im_r * im_t)
                              + im_h * (re_r * im_t - im_r * re_t))

            acc = lax.fori_loop(0, HALF, dim_body,
                                jnp.zeros((LANES,), jnp.float32), unroll=4)
            outv[pl.ds(cbase + g * LANES, LANES)] = acc
            return carry

        lax.fori_loop(0, GROUPS, group_body, 0)

    pltpu.sync_copy(outv, out_hbm.at[pl.ds(base, BPW)])


def kernel(head_index, relation_index, tail_index, time_index,
           entity_embedding, relation_embedding, time_embedding,
           aux_embedding):
    del time_index, time_embedding, aux_embedding
    return _kge_score(entity_embedding, relation_embedding,
                      head_index.astype(jnp.int32),
                      tail_index.astype(jnp.int32),
                      relation_index.astype(jnp.int32))


# speedup vs baseline: 1.5956x; 1.0280x over previous
import functools

import jax
import jax.numpy as jnp
from jax import lax
from jax.experimental import pallas as pl
from jax.experimental.pallas import tpu as pltpu
from jax.experimental.pallas import tpu_sc as plsc

BATCH = 16384
DIM = 64
HALF = DIM // 2
LANES = 16
NREL = 230
NUM_WORKERS = 32
BPW = BATCH // NUM_WORKERS
NCH = 2
CH = BPW // NCH
K = 16
NB = CH // K
GROUPS = CH // LANES

_mesh = plsc.VectorSubcoreMesh(core_axis_name="c", subcore_axis_name="s")


@functools.partial(
    pl.kernel,
    mesh=_mesh,
    out_type=jax.ShapeDtypeStruct((BATCH,), jnp.float32),
    scratch_types=[
        pltpu.VMEM((BPW,), jnp.int32),
        pltpu.VMEM((BPW,), jnp.int32),
        pltpu.VMEM((BPW,), jnp.int32),
        pltpu.VMEM((CH, DIM), jnp.float32),
        pltpu.VMEM((CH, DIM), jnp.float32),
        pltpu.VMEM((NREL, DIM), jnp.float32),
        pltpu.VMEM((BPW,), jnp.float32),
        pltpu.SemaphoreType.DMA,
        pltpu.SemaphoreType.DMA,
        pltpu.SemaphoreType.DMA,
    ],
    compiler_params=pltpu.CompilerParams(needs_layout_passes=False),
)
def _kge_score(ent_hbm, rel_hbm, hidx_hbm, tidx_hbm, ridx_hbm, out_hbm,
               hidx_v, tidx_v, ridx_v, hrow, trow, rtab, outv,
               semA, semB, semR):
    wid = lax.axis_index("s") * 2 + lax.axis_index("c")
    base = wid * BPW

    pltpu.sync_copy(hidx_hbm.at[pl.ds(base, BPW)], hidx_v)
    pltpu.sync_copy(tidx_hbm.at[pl.ds(base, BPW)], tidx_v)
    pltpu.sync_copy(ridx_hbm.at[pl.ds(base, BPW)], ridx_v)
    rel_cp = pltpu.async_copy(rel_hbm, rtab, semR)

    def issue_burst(cbase, j, sem):
        i0 = j * K
        hvec = hidx_v[pl.ds(cbase + i0, K)]
        tvec = tidx_v[pl.ds(cbase + i0, K)]
        for k in range(K):
            i = i0 + k
            pltpu.async_copy(ent_hbm.at[pl.ds(hvec[k], 1), :],
                             hrow.at[pl.ds(i, 1), :], sem)
            pltpu.async_copy(ent_hbm.at[pl.ds(tvec[k], 1), :],
                             trow.at[pl.ds(i, 1), :], sem)

    def drain_burst(sem):
        for k in range(K):
            pltpu.make_async_copy(ent_hbm.at[pl.ds(0, 1), :],
                                  hrow.at[pl.ds(k, 1), :], sem).wait()
            pltpu.make_async_copy(ent_hbm.at[pl.ds(0, 1), :],
                                  trow.at[pl.ds(k, 1), :], sem).wait()

    lane_iota = lax.iota(jnp.int32, LANES)

    for c in range(NCH):
        cbase = c * CH

        def pair_body(j, carry):
            @pl.when(j > 0)
            def _():
                drain_burst(semA)
            issue_burst(cbase, 2 * j, semA)

            @pl.when(j > 0)
            def _():
                drain_burst(semB)
            issue_burst(cbase, 2 * j + 1, semB)
            return carry

        lax.fori_loop(0, NB // 2, pair_body, 0)
        drain_burst(semA)
        drain_burst(semB)
        if c == 0:
            rel_cp.wait()

        def group_body(g, carry):
            rows = g * LANES + lane_iota
            rrows = ridx_v[pl.ds(cbase + g * LANES, LANES)]

            def dim_body(d, acc):
                cre = jnp.broadcast_to(d, (LANES,))
                cim = cre + HALF
                re_h = plsc.load_gather(hrow, [rows, cre])
                im_h = plsc.load_gather(hrow, [rows, cim])
                re_r = plsc.load_gather(rtab, [rrows, cre])
                im_r = plsc.load_gather(rtab, [rrows, cim])
                re_t = plsc.load_gather(trow, [rows, cre])
                im_t = plsc.load_gather(trow, [rows, cim])
                return acc + (re_h * (re_r * re_t + im_r * im_t)
                              + im_h * (re_r * im_t - im_r * re_t))

            acc = lax.fori_loop(0, HALF, dim_body,
                                jnp.zeros((LANES,), jnp.float32), unroll=4)
            outv[pl.ds(cbase + g * LANES, LANES)] = acc
            return carry

        lax.fori_loop(0, GROUPS, group_body, 0)

    pltpu.sync_copy(outv, out_hbm.at[pl.ds(base, BPW)])


def kernel(head_index, relation_index, tail_index, time_index,
           entity_embedding, relation_embedding, time_embedding,
           aux_embedding):
    del time_index, time_embedding, aux_embedding
    return _kge_score(entity_embedding, relation_embedding,
                      head_index.astype(jnp.int32),
                      tail_index.astype(jnp.int32),
                      relation_index.astype(jnp.int32))


# speedup vs baseline: 1.7643x; 1.1057x over previous
import functools

import jax
import jax.numpy as jnp
from jax import lax
from jax.experimental import pallas as pl
from jax.experimental.pallas import tpu as pltpu
from jax.experimental.pallas import tpu_sc as plsc

BATCH = 16384
NENT = 1000000
DIM = 64
HALF = DIM // 2
LANES = 16
NREL = 230
NW = 32
RANGE = 31232
ALIGNED_END = 999936
NTAIL = NENT - ALIGNED_END
E = 512
NCHUNK = 62
MAXSTART = ALIGNED_END - E
LISTCAP = 2080
PACK = 64
NSTAGE = 2 * BATCH
STAGE = NSTAGE + PACK

BPW = BATCH // NW
NCH2 = 2
CH2 = BPW // NCH2
GROUPS2 = CH2 // LANES

_mesh = plsc.VectorSubcoreMesh(core_axis_name="c", subcore_axis_name="s")


@functools.partial(
    pl.kernel,
    mesh=_mesh,
    out_type=jax.ShapeDtypeStruct((STAGE, 2 * DIM), jnp.float32),
    scratch_types=[
        pltpu.VMEM((2048,), jnp.int32),
        pltpu.VMEM((LISTCAP,), jnp.int32),
        pltpu.VMEM((LISTCAP,), jnp.int32),
        pltpu.VMEM((DIM, E), jnp.float32),
        pltpu.VMEM((DIM, E), jnp.float32),
        pltpu.VMEM((PACK, 2 * DIM), jnp.float32),
        pltpu.VMEM((PACK, 2 * DIM), jnp.float32),
        pltpu.VMEM((PACK,), jnp.int32),
        pltpu.VMEM((PACK,), jnp.int32),
        pltpu.VMEM((PACK,), jnp.int32),
        pltpu.VMEM((NTAIL, DIM), jnp.float32),
        pltpu.SemaphoreType.DMA,
        pltpu.SemaphoreType.DMA,
        pltpu.SemaphoreType.DMA,
        pltpu.SemaphoreType.DMA,
    ],
    compiler_params=pltpu.CompilerParams(needs_layout_passes=False),
)
def _phase1(entT_hbm, tail_hbm, hidx_hbm, tidx_hbm, stage_hbm,
            piece, entlist, poslist, bufA, bufB, packA, packB,
            posA, posB, hitent, tailtab, semSA, semSB, semWA, semWB):
    wid = lax.axis_index("s") * 2 + lax.axis_index("c")
    lo = wid * RANGE
    hi = jnp.where(wid == NW - 1, NENT, lo + RANGE)
    lane_iota = lax.iota(jnp.int32, LANES)

    def chunk_start(c):
        return jnp.minimum(lo + c * E, MAXSTART)

    cpA0 = pltpu.async_copy(
        entT_hbm.at[:, pl.ds(chunk_start(0), E)], bufA, semSA)
    cpB0 = pltpu.async_copy(
        entT_hbm.at[:, pl.ds(chunk_start(1), E)], bufB, semSB)
    del cpA0, cpB0
    pltpu.sync_copy(tail_hbm, tailtab)

    def init_list(i, carry):
        entlist[pl.ds(i * LANES, LANES)] = jnp.full((LANES,), -1, jnp.int32)
        return carry
    lax.fori_loop(0, LISTCAP // LANES, init_list, 0)

    def scan_src(arr_hbm, pos0, cnt_in):
        def piece_body(p, cnt):
            pltpu.sync_copy(arr_hbm.at[pl.ds(p * 2048, 2048)], piece)

            def vec_body(i, cnt):
                v = piece[pl.ds(i * LANES, LANES)]
                m = (v >= lo) & (v < hi)
                cc = jnp.minimum(cnt, LISTCAP - LANES)
                plsc.store_compressed(entlist.at[pl.ds(cc, LANES)], v, mask=m)
                pos = pos0 + p * 2048 + i * LANES + lane_iota
                plsc.store_compressed(poslist.at[pl.ds(cc, LANES)], pos, mask=m)
                npop = plsc.all_reduce_population_count(m)
                return cnt + npop[0]

            return lax.fori_loop(0, 2048 // LANES, vec_body, cnt)
        return lax.fori_loop(0, BATCH // 2048, piece_body, cnt_in)

    cnt = scan_src(hidx_hbm, 0, jnp.int32(0))
    cnt = scan_src(tidx_hbm, BATCH, cnt)

    def drain_stream(sem, buf):
        pltpu.make_async_copy(
            entT_hbm.at[:, pl.ds(0, E)], buf, sem).wait()

    def drain_scatter(sem, pack):
        pltpu.make_async_copy(
            stage_hbm.at[pl.ds(0, PACK), :], pack, sem).wait()

    def extract_chunk(e0, e1, col_of, src_gather, pack, posb, semW):
        for q in range(PACK // LANES):
            posb[pl.ds(q * LANES, LANES)] = (
                NSTAGE + q * LANES + lane_iota)

        def list_body(i, hcnt):
            ev = entlist[pl.ds(i * LANES, LANES)]
            m = (ev >= e0) & (ev < e1)
            pv = poslist[pl.ds(i * LANES, LANES)]
            hc = jnp.minimum(hcnt, PACK - LANES)
            plsc.store_compressed(hitent.at[pl.ds(hc, LANES)], ev, mask=m)
            plsc.store_compressed(posb.at[pl.ds(hc, LANES)], pv, mask=m)
            return hcnt + plsc.all_reduce_population_count(m)[0]

        hcnt = lax.fori_loop(0, LISTCAP // LANES, list_body, jnp.int32(0))
        hcnt = jnp.minimum(hcnt, PACK)

        def member_body(j, carry):
            jsplat = jnp.broadcast_to(j, (LANES,))
            e = plsc.load_gather(hitent, [jsplat])[0]
            col = col_of(e)
            for q in range(DIM // LANES):
                seg = src_gather(q, col)
                plsc.store_scatter(
                    pack, [jsplat, q * LANES + lane_iota], seg)
            return carry

        lax.fori_loop(0, hcnt, member_body, 0)
        return pltpu.async_copy(pack, stage_hbm.at[posb], semW)

    def pair_body(j, carry):
        e0 = lo + (2 * j) * E
        s0 = chunk_start(2 * j)
        drain_stream(semSA, bufA)

        @pl.when(j > 0)
        def _():
            drain_scatter(semWA, packA)

        def gatherA(q, col):
            return plsc.load_gather(
                bufA, [q * LANES + lane_iota, jnp.broadcast_to(col, (LANES,))])

        extract_chunk(e0, e0 + E, lambda e: e - s0, gatherA,
                      packA, posA, semWA)
        pltpu.async_copy(
            entT_hbm.at[:, pl.ds(chunk_start(2 * j + 2), E)], bufA, semSA)

        e0b = lo + (2 * j + 1) * E
        s0b = chunk_start(2 * j + 1)
        drain_stream(semSB, bufB)

        @pl.when(j > 0)
        def _():
            drain_scatter(semWB, packB)

        def gatherB(q, col):
            return plsc.load_gather(
                bufB, [q * LANES + lane_iota, jnp.broadcast_to(col, (LANES,))])

        extract_chunk(e0b, e0b + E, lambda e: e - s0b, gatherB,
                      packB, posB, semWB)
        pltpu.async_copy(
            entT_hbm.at[:, pl.ds(chunk_start(2 * j + 3), E)], bufB, semSB)
        return carry

    lax.fori_loop(0, NCHUNK // 2, pair_body, 0)

    drain_scatter(semWA, packA)

    def gatherT(q, row):
        return plsc.load_gather(
            tailtab, [jnp.broadcast_to(row, (LANES,)), q * LANES + lane_iota])

    extract_chunk(ALIGNED_END, NENT, lambda e: e - ALIGNED_END, gatherT,
                  packA, posA, semWA)

    drain_scatter(semWA, packA)
    drain_scatter(semWB, packB)
    drain_stream(semSA, bufA)
    drain_stream(semSB, bufB)


@functools.partial(
    pl.kernel,
    mesh=_mesh,
    out_type=jax.ShapeDtypeStruct((BATCH,), jnp.float32),
    scratch_types=[
        pltpu.VMEM((BPW,), jnp.int32),
        pltpu.VMEM((CH2, 2 * DIM), jnp.float32),
        pltpu.VMEM((CH2, 2 * DIM), jnp.float32),
        pltpu.VMEM((NREL, DIM), jnp.float32),
        pltpu.VMEM((BPW,), jnp.float32),
        pltpu.SemaphoreType.DMA,
    ],
    compiler_params=pltpu.CompilerParams(needs_layout_passes=False),
)
def _phase2(stage_hbm, rel_hbm, ridx_hbm, out_hbm,
            ridx_v, hrow, trow, rtab, outv, sem):
    wid = lax.axis_index("s") * 2 + lax.axis_index("c")
    base = wid * BPW

    pltpu.sync_copy(ridx_hbm.at[pl.ds(base, BPW)], ridx_v)
    pltpu.sync_copy(rel_hbm, rtab)

    lane_iota = lax.iota(jnp.int32, LANES)

    for c in range(NCH2):
        cbase = c * CH2
        cph = pltpu.async_copy(
            stage_hbm.at[pl.ds(base + cbase, CH2), :], hrow, sem)
        cpt = pltpu.async_copy(
            stage_hbm.at[pl.ds(BATCH + base + cbase, CH2), :], trow, sem)
        cph.wait()
        cpt.wait()

        def group_body(g, carry):
            rows = g * LANES + lane_iota
            rrows = ridx_v[pl.ds(cbase + g * LANES, LANES)]

            def dim_body(d, acc):
                cre = jnp.broadcast_to(d, (LANES,))
                cim = cre + HALF
                re_h = plsc.load_gather(hrow, [rows, cre])
                im_h = plsc.load_gather(hrow, [rows, cim])
                re_r = plsc.load_gather(rtab, [rrows, cre])
                im_r = plsc.load_gather(rtab, [rrows, cim])
                re_t = plsc.load_gather(trow, [rows, cre])
                im_t = plsc.load_gather(trow, [rows, cim])
                return acc + (re_h * (re_r * re_t + im_r * im_t)
                              + im_h * (re_r * im_t - im_r * re_t))

            acc = lax.fori_loop(0, HALF, dim_body,
                                jnp.zeros((LANES,), jnp.float32), unroll=4)
            outv[pl.ds(cbase + g * LANES, LANES)] = acc
            return carry

        lax.fori_loop(0, GROUPS2, group_body, 0)

    pltpu.sync_copy(outv, out_hbm.at[pl.ds(base, BPW)])


def kernel(head_index, relation_index, tail_index, time_index,
           entity_embedding, relation_embedding, time_embedding,
           aux_embedding):
    del time_index, time_embedding, aux_embedding
    hidx = head_index.astype(jnp.int32)
    tidx = tail_index.astype(jnp.int32)
    ridx = relation_index.astype(jnp.int32)
    tail_tbl = entity_embedding[ALIGNED_END:, :]
    staging = _phase1(entity_embedding.T, tail_tbl, hidx, tidx)
    return _phase2(staging, relation_embedding, ridx)
